# Initial kernel scaffold; baseline (speedup 1.0000x reference)
import jax
import jax.numpy as jnp
from jax.experimental import pallas as pl


def kernel(x, edge_index, W, b):
    raise NotImplementedError("write your pallas kernel here")


# speedup vs baseline: 17.5130x; 17.5130x over previous
import jax
import jax.numpy as jnp
from jax import lax
from jax.experimental import pallas as pl
from jax.experimental.pallas import tpu as pltpu, tpu_sc as plsc

N_NODES = 100000
IN_CH = 25
OUT_CH = 50
HALF = 16

NC = 2
NS = 16
B = 128
J = 8
G = 196
CHUNK = J * B
E_TILE = CHUNK * G
E_PAD = NS * E_TILE
ROWS_PER_TILE = 6272
N_ACC = NS * ROWS_PER_TILE


def _sc_body(src_hbm, dst_hbm, tab_hbm, out_hbm, idx_s, idx_d, rows, acc, sem):
    c = lax.axis_index("c")
    s = lax.axis_index("s")

    def zero_row(i, carry):
        rows[i, :] = jnp.zeros((16,), jnp.float32)
        return carry

    lax.fori_loop(0, CHUNK, zero_row, 0)
    base = s * ROWS_PER_TILE
    for k in range(ROWS_PER_TILE // CHUNK):
        pltpu.sync_copy(rows, acc.at[pl.ds(base + k * CHUNK, CHUNK)])
    tail = ROWS_PER_TILE % CHUNK
    if tail:
        pltpu.sync_copy(rows.at[pl.ds(0, tail)],
                        acc.at[pl.ds(base + ROWS_PER_TILE - tail, tail)])
    plsc.subcore_barrier()

    def step(g, carry):
        pltpu.sync_copy(src_hbm.at[c, s, g], idx_s)
        pltpu.sync_copy(dst_hbm.at[s, g], idx_d)
        handles = [
            pltpu.async_copy(tab_hbm.at[idx_s.at[j]],
                             rows.at[pl.ds(j * B, B)], sem)
            for j in range(J)
        ]
        for h in handles:
            h.wait()
        for j in range(J):
            pltpu.sync_copy(rows.at[pl.ds(j * B, B)],
                            acc.at[idx_d.at[j]], add=True)
        return carry

    lax.fori_loop(0, G, step, 0)
    plsc.subcore_barrier()

    for k in range(ROWS_PER_TILE // CHUNK):
        pltpu.sync_copy(acc.at[pl.ds(base + k * CHUNK, CHUNK)],
                        out_hbm.at[c, pl.ds(base + k * CHUNK, CHUNK)])
    if tail:
        pltpu.sync_copy(acc.at[pl.ds(base + ROWS_PER_TILE - tail, tail)],
                        out_hbm.at[c, pl.ds(base + ROWS_PER_TILE - tail, tail)])


_sc_agg = pl.kernel(
    _sc_body,
    out_type=jax.ShapeDtypeStruct((NC, N_ACC, HALF), jnp.float32),
    mesh=plsc.VectorSubcoreMesh(core_axis_name="c", subcore_axis_name="s"),
    scratch_types=[
        pltpu.VMEM((J, B), jnp.int32),
        pltpu.VMEM((J, B), jnp.int32),
        pltpu.VMEM((CHUNK, HALF), jnp.float32),
        pltpu.VMEM_SHARED((N_ACC, HALF), jnp.float32),
        pltpu.SemaphoreType.DMA,
    ],
    compiler_params=pltpu.CompilerParams(use_tc_tiling_on_sc=False),
)


def _mm_body(a0_ref, a1_ref, w0_ref, w1_ref, b_ref, o_ref):
    acc = jnp.dot(a0_ref[...], w0_ref[...], preferred_element_type=jnp.float32)
    acc = acc + jnp.dot(a1_ref[...], w1_ref[...],
                        preferred_element_type=jnp.float32)
    o_ref[...] = jnp.maximum(acc + b_ref[...], 0.0)


_BM = 1024

_mm = pl.pallas_call(
    _mm_body,
    grid=(N_ACC // _BM,),
    in_specs=[
        pl.BlockSpec((_BM, HALF), lambda i: (i, 0)),
        pl.BlockSpec((_BM, HALF), lambda i: (i, 0)),
        pl.BlockSpec((HALF, OUT_CH), lambda i: (0, 0)),
        pl.BlockSpec((HALF, OUT_CH), lambda i: (0, 0)),
        pl.BlockSpec((1, OUT_CH), lambda i: (0, 0)),
    ],
    out_specs=pl.BlockSpec((_BM, OUT_CH), lambda i: (i, 0)),
    out_shape=jax.ShapeDtypeStruct((N_ACC, OUT_CH), jnp.float32),
)


def kernel(x, edge_index, W, b):
    x = x.astype(jnp.float32)
    src = edge_index[0].astype(jnp.int32)
    dst = edge_index[1].astype(jnp.int32)

    tab = jnp.concatenate(
        [x[:, :HALF], jnp.pad(x[:, HALF:], ((0, 0), (0, 2 * HALF - IN_CH)))],
        axis=0)

    pad = E_PAD - src.shape[0]
    src_p = jnp.concatenate([src, jnp.zeros((pad,), jnp.int32)])
    garbage = N_NODES + (jnp.arange(pad, dtype=jnp.int32) % (N_ACC - N_NODES))
    dst_p = jnp.concatenate([dst, garbage])

    src2 = jnp.stack([src_p, src_p + N_NODES]).reshape(NC, NS, G, J, B)
    dst4 = dst_p.reshape(NS, G, J, B)

    agg = _sc_agg(src2, dst4, tab)

    w0 = W[:HALF].astype(jnp.float32)
    w1 = jnp.pad(W[HALF:].astype(jnp.float32),
                 ((0, 2 * HALF - IN_CH), (0, 0)))
    out = _mm(agg[0], agg[1], w0, w1, b.reshape(1, OUT_CH).astype(jnp.float32))
    return out[:N_NODES]


# speedup vs baseline: 18.8494x; 1.0763x over previous
import jax
import jax.numpy as jnp
from jax import lax
from jax.experimental import pallas as pl
from jax.experimental.pallas import tpu as pltpu, tpu_sc as plsc

N_NODES = 100000
IN_CH = 25
OUT_CH = 50
HALF = 16

NC = 2
NS = 16
B = 128
J = 8
G = 196
CHUNK = J * B
E_TILE = CHUNK * G
E_PAD = NS * E_TILE
ROWS_PER_TILE = 6272
N_ACC = NS * ROWS_PER_TILE


def _sc_body(src_hbm, dst_hbm, tab_hbm, out_hbm, idx_s, idx_d, rows, acc, sem,
             sem2):
    c = lax.axis_index("c")
    s = lax.axis_index("s")

    def zero_row(i, carry):
        rows[i, :] = jnp.zeros((16,), jnp.float32)
        return carry

    lax.fori_loop(0, CHUNK, zero_row, 0)
    base = s * ROWS_PER_TILE
    for k in range(ROWS_PER_TILE // CHUNK):
        pltpu.sync_copy(rows, acc.at[pl.ds(base + k * CHUNK, CHUNK)])
    tail = ROWS_PER_TILE % CHUNK
    if tail:
        pltpu.sync_copy(rows.at[pl.ds(0, tail)],
                        acc.at[pl.ds(base + ROWS_PER_TILE - tail, tail)])
    plsc.subcore_barrier()

    def step(g, carry):
        pltpu.sync_copy(src_hbm.at[c, s, g], idx_s)
        pltpu.sync_copy(dst_hbm.at[s, g], idx_d)
        handles = [
            pltpu.async_copy(tab_hbm.at[idx_s.at[j]],
                             rows.at[pl.ds(j * B, B)], sem)
            for j in range(J)
        ]
        for h in handles:
            h.wait()
        shandles = [
            pltpu.async_copy(rows.at[pl.ds(j * B, B)],
                             acc.at[idx_d.at[j]], sem2, add=True)
            for j in range(J)
        ]
        for h in shandles:
            h.wait()
        return carry

    lax.fori_loop(0, G, step, 0)
    plsc.subcore_barrier()

    for k in range(ROWS_PER_TILE // CHUNK):
        pltpu.sync_copy(acc.at[pl.ds(base + k * CHUNK, CHUNK)],
                        out_hbm.at[c, pl.ds(base + k * CHUNK, CHUNK)])
    if tail:
        pltpu.sync_copy(acc.at[pl.ds(base + ROWS_PER_TILE - tail, tail)],
                        out_hbm.at[c, pl.ds(base + ROWS_PER_TILE - tail, tail)])


_sc_agg = pl.kernel(
    _sc_body,
    out_type=jax.ShapeDtypeStruct((NC, N_ACC, HALF), jnp.float32),
    mesh=plsc.VectorSubcoreMesh(core_axis_name="c", subcore_axis_name="s"),
    scratch_types=[
        pltpu.VMEM((J, B), jnp.int32),
        pltpu.VMEM((J, B), jnp.int32),
        pltpu.VMEM((CHUNK, HALF), jnp.float32),
        pltpu.VMEM_SHARED((N_ACC, HALF), jnp.float32),
        pltpu.SemaphoreType.DMA,
        pltpu.SemaphoreType.DMA,
    ],
    compiler_params=pltpu.CompilerParams(use_tc_tiling_on_sc=False),
)


def _mm_body(a0_ref, a1_ref, w0_ref, w1_ref, b_ref, o_ref):
    acc = jnp.dot(a0_ref[...], w0_ref[...], preferred_element_type=jnp.float32)
    acc = acc + jnp.dot(a1_ref[...], w1_ref[...],
                        preferred_element_type=jnp.float32)
    o_ref[...] = jnp.maximum(acc + b_ref[...], 0.0)


_BM = 1024

_mm = pl.pallas_call(
    _mm_body,
    grid=(N_ACC // _BM,),
    in_specs=[
        pl.BlockSpec((_BM, HALF), lambda i: (i, 0)),
        pl.BlockSpec((_BM, HALF), lambda i: (i, 0)),
        pl.BlockSpec((HALF, OUT_CH), lambda i: (0, 0)),
        pl.BlockSpec((HALF, OUT_CH), lambda i: (0, 0)),
        pl.BlockSpec((1, OUT_CH), lambda i: (0, 0)),
    ],
    out_specs=pl.BlockSpec((_BM, OUT_CH), lambda i: (i, 0)),
    out_shape=jax.ShapeDtypeStruct((N_ACC, OUT_CH), jnp.float32),
)


def kernel(x, edge_index, W, b):
    x = x.astype(jnp.float32)
    src = edge_index[0].astype(jnp.int32)
    dst = edge_index[1].astype(jnp.int32)

    tab = jnp.concatenate(
        [x[:, :HALF], jnp.pad(x[:, HALF:], ((0, 0), (0, 2 * HALF - IN_CH)))],
        axis=0)

    pad = E_PAD - src.shape[0]
    src_p = jnp.concatenate([src, jnp.zeros((pad,), jnp.int32)])
    garbage = N_NODES + (jnp.arange(pad, dtype=jnp.int32) % (N_ACC - N_NODES))
    dst_p = jnp.concatenate([dst, garbage])

    src2 = jnp.stack([src_p, src_p + N_NODES]).reshape(NC, NS, G, J, B)
    dst4 = dst_p.reshape(NS, G, J, B)

    agg = _sc_agg(src2, dst4, tab)

    w0 = W[:HALF].astype(jnp.float32)
    w1 = jnp.pad(W[HALF:].astype(jnp.float32),
                 ((0, 2 * HALF - IN_CH), (0, 0)))
    out = _mm(agg[0], agg[1], w0, w1, b.reshape(1, OUT_CH).astype(jnp.float32))
    return out[:N_NODES]
